# baseline (speedup 1.0000x reference)
import jax
import jax.numpy as jnp
from jax import lax
from jax.experimental import pallas as pl
from jax.experimental.pallas import tpu as pltpu
from jax.experimental.pallas import tpu_sc as plsc

BINS = 256
NC = 2
NS = 16
L = 16
NW = NC * NS
RPC = 32
G = 8


def _body(x_hbm, out_hbm, xbuf0, xbuf1, obuf0, obuf1, hist, cdf,
          sem0, sem1, osem0, osem1):
    wid = lax.axis_index("s") * NC + lax.axis_index("c")
    lane = lax.iota(jnp.int32, L)
    lane_off = lane * BINS
    rows = x_hbm.shape[1]
    w = x_hbm.shape[2]
    vpr = w // L
    n_chunks = rows // RPC
    bufs = (xbuf0, xbuf1)
    sems = (sem0, sem1)
    obufs = (obuf0, obuf1)
    osems = (osem0, osem1)

    def stream_in(img, compute, carry_init):
        pltpu.async_copy(x_hbm.at[img, pl.ds(0, RPC)], bufs[0], sems[0])

        def outer(c2, carry):
            for b in range(2):
                c = c2 * 2 + b
                nb = (b + 1) % 2

                @pl.when(c + 1 < n_chunks)
                def _():
                    pltpu.async_copy(
                        x_hbm.at[img, pl.ds((c + 1) * RPC, RPC)],
                        bufs[nb], sems[nb])

                pltpu.make_async_copy(
                    x_hbm.at[img, pl.ds(c * RPC, RPC)], bufs[b], sems[b]).wait()
                carry = compute(c, bufs[b], carry)
            return carry

        return lax.fori_loop(0, n_chunks // 2, outer, carry_init)

    for rr in range(x_hbm.shape[0] // NW):
        img = wid * (x_hbm.shape[0] // NW) + rr

        def mm_chunk(c, buf, carry):
            @plsc.parallel_loop(0, RPC, carry=carry)
            def mm(r, carry2):
                vmn2, vmx2 = carry2
                vs = [buf[r, pl.ds(k * L, L)] for k in range(vpr)]
                lo, hi = vs, vs
                while len(lo) > 1:
                    lo = [jnp.minimum(a, b) for a, b in zip(lo[::2], lo[1::2])]
                    hi = [jnp.maximum(a, b) for a, b in zip(hi[::2], hi[1::2])]
                return (jnp.minimum(vmn2, lo[0]), jnp.maximum(vmx2, hi[0]))

            return mm

        vmn0 = jnp.full((L,), jnp.inf, jnp.float32)
        vmx0 = jnp.full((L,), -jnp.inf, jnp.float32)
        vmn, vmx = stream_in(img, mm_chunk, (vmn0, vmx0))
        mn = -plsc.cummax(-vmn)[L - 1]
        mx = plsc.cummax(vmx)[L - 1]
        scale_v = jnp.full((L,), float(BINS - 1), jnp.float32) / (mx - mn + 1e-8)
        scale = scale_v[0]

        zero_v = jnp.zeros((L,), jnp.float32)

        @plsc.parallel_loop(0, BINS * L, step=L)
        def zrow(j):
            hist[pl.ds(j, L)] = zero_v

        ones_v = jnp.ones((L,), jnp.float32)

        def hist_chunk(c, buf, carry):
            @plsc.parallel_loop(0, RPC)
            def vec_h(r):
                for g in range(vpr // G):
                    vs = [buf[r, pl.ds((g * G + k) * L, L)] for k in range(G)]
                    ts = [v - mn for v in vs]
                    ts = [t * scale for t in ts]
                    ts = [jnp.minimum(t, float(BINS - 1)) for t in ts]
                    ids = [t.astype(jnp.int32) for t in ts]
                    ads = [i + lane_off for i in ids]
                    for a in ads:
                        plsc.addupdate_scatter(hist, [a], ones_v)

            return carry

        stream_in(img, hist_chunk, 0)

        def grp(j2, tot):
            acc = zero_v
            for k in range(L):
                acc = acc + plsc.load_gather(hist, [lane + (k * BINS + j2 * L)])
            c = plsc.cumsum(acc) + tot
            cdf[pl.ds(j2 * L, L)] = c
            return c[L - 1]

        tot = lax.fori_loop(0, BINS // L, grp, jnp.float32(0.0))
        c0 = cdf[pl.ds(0, L)][0]
        inv = (jnp.ones((L,), jnp.float32) / (tot - c0 + 1e-8))[0]

        @plsc.parallel_loop(0, BINS, step=L)
        def nrm(j2):
            v = cdf[pl.ds(j2, L)]
            cdf[pl.ds(j2, L)] = (v - c0) * inv

        def eq_outer(c2, _):
            for b in range(2):
                c = c2 * 2 + b
                nb = (b + 1) % 2
                ob = obufs[b]

                @pl.when(c + 1 < n_chunks)
                def _():
                    pltpu.async_copy(
                        x_hbm.at[img, pl.ds((c + 1) * RPC, RPC)],
                        bufs[nb], sems[nb])

                pltpu.make_async_copy(
                    x_hbm.at[img, pl.ds(c * RPC, RPC)], bufs[b], sems[b]).wait()

                @pl.when(c2 > 0)
                def _():
                    pltpu.make_async_copy(
                        ob, out_hbm.at[img, pl.ds(c * RPC, RPC)],
                        osems[b]).wait()

                buf = bufs[b]

                @plsc.parallel_loop(0, RPC)
                def vec_e(r):
                    for g in range(vpr // G):
                        vs = [buf[r, pl.ds((g * G + k) * L, L)]
                              for k in range(G)]
                        ts = [v - mn for v in vs]
                        ts = [t * scale for t in ts]
                        ts = [jnp.minimum(t, float(BINS - 1)) for t in ts]
                        ids = [t.astype(jnp.int32) for t in ts]
                        res = [plsc.load_gather(cdf, [i]) for i in ids]
                        for k in range(G):
                            ob[r, pl.ds((g * G + k) * L, L)] = res[k]

                pltpu.async_copy(ob, out_hbm.at[img, pl.ds(c * RPC, RPC)],
                                 osems[b])
            return 0

        pltpu.async_copy(x_hbm.at[img, pl.ds(0, RPC)], bufs[0], sems[0])
        lax.fori_loop(0, n_chunks // 2, eq_outer, 0)
        for b in range(2):
            pltpu.make_async_copy(
                obufs[b],
                out_hbm.at[img, pl.ds((n_chunks - 2 + b) * RPC, RPC)],
                osems[b]).wait()


def kernel(x):
    b, h, w = x.shape
    mesh = plsc.VectorSubcoreMesh(core_axis_name="c", subcore_axis_name="s")
    run = pl.kernel(
        _body,
        out_type=jax.ShapeDtypeStruct((b, h, w), jnp.float32),
        mesh=mesh,
        compiler_params=pltpu.CompilerParams(
            needs_layout_passes=False, use_tc_tiling_on_sc=True),
        scratch_types=[
            pltpu.VMEM((RPC, w), jnp.float32),
            pltpu.VMEM((RPC, w), jnp.float32),
            pltpu.VMEM((RPC, w), jnp.float32),
            pltpu.VMEM((RPC, w), jnp.float32),
            pltpu.VMEM((BINS * L,), jnp.float32),
            pltpu.VMEM((BINS,), jnp.float32),
            pltpu.SemaphoreType.DMA,
            pltpu.SemaphoreType.DMA,
            pltpu.SemaphoreType.DMA,
            pltpu.SemaphoreType.DMA,
        ],
    )
    return run(x)


# speedup vs baseline: 1.2048x; 1.2048x over previous
import jax
import jax.numpy as jnp
from jax import lax
from jax.experimental import pallas as pl
from jax.experimental.pallas import tpu as pltpu
from jax.experimental.pallas import tpu_sc as plsc

BINS = 256
NC = 2
NS = 16
L = 16
NW = NC * NS
RPC = 32
OPC = 16
G = 8


def _body(x_hbm, out_hbm, xbuf0, xbuf1, obuf0, obuf1, idxc, hist, cdf,
          sem0, sem1, osem0, osem1):
    wid = lax.axis_index("s") * NC + lax.axis_index("c")
    lane = lax.iota(jnp.int32, L)
    lane_off = lane * BINS
    rows = x_hbm.shape[1]
    w = x_hbm.shape[2]
    vpr = w // L
    n_chunks = rows // RPC
    bufs = (xbuf0, xbuf1)
    sems = (sem0, sem1)
    obufs = (obuf0, obuf1)
    osems = (osem0, osem1)

    def stream_in(img, compute, carry_init):
        pltpu.async_copy(x_hbm.at[img, pl.ds(0, RPC)], bufs[0], sems[0])

        def outer(c2, carry):
            for b in range(2):
                c = c2 * 2 + b
                nb = (b + 1) % 2

                @pl.when(c + 1 < n_chunks)
                def _():
                    pltpu.async_copy(
                        x_hbm.at[img, pl.ds((c + 1) * RPC, RPC)],
                        bufs[nb], sems[nb])

                pltpu.make_async_copy(
                    x_hbm.at[img, pl.ds(c * RPC, RPC)], bufs[b], sems[b]).wait()
                carry = compute(c, bufs[b], carry)
            return carry

        return lax.fori_loop(0, n_chunks // 2, outer, carry_init)

    for rr in range(x_hbm.shape[0] // NW):
        img = wid * (x_hbm.shape[0] // NW) + rr

        def mm_chunk(c, buf, carry):
            @plsc.parallel_loop(0, RPC, carry=carry)
            def mm(r, carry2):
                vmn2, vmx2 = carry2
                vs = [buf[r, pl.ds(k * L, L)] for k in range(vpr)]
                lo, hi = vs, vs
                while len(lo) > 1:
                    lo = [jnp.minimum(a, b) for a, b in zip(lo[::2], lo[1::2])]
                    hi = [jnp.maximum(a, b) for a, b in zip(hi[::2], hi[1::2])]
                return (jnp.minimum(vmn2, lo[0]), jnp.maximum(vmx2, hi[0]))

            return mm

        vmn0 = jnp.full((L,), jnp.inf, jnp.float32)
        vmx0 = jnp.full((L,), -jnp.inf, jnp.float32)
        vmn, vmx = stream_in(img, mm_chunk, (vmn0, vmx0))
        mn = -plsc.cummax(-vmn)[L - 1]
        mx = plsc.cummax(vmx)[L - 1]
        scale_v = jnp.full((L,), float(BINS - 1), jnp.float32) / (mx - mn + 1e-8)
        scale = scale_v[0]

        zero_v = jnp.zeros((L,), jnp.float32)

        @plsc.parallel_loop(0, BINS * L, step=L)
        def zrow(j):
            hist[pl.ds(j, L)] = zero_v

        ones_v = jnp.ones((L,), jnp.float32)

        def hist_chunk(c, buf, carry):
            @plsc.parallel_loop(0, RPC)
            def vec_h(r):
                for g in range(vpr // G):
                    vs = [buf[r, pl.ds((g * G + k) * L, L)] for k in range(G)]
                    ts = [v - mn for v in vs]
                    ts = [t * scale for t in ts]
                    ts = [jnp.minimum(t, float(BINS - 1)) for t in ts]
                    ids = [t.astype(jnp.int32) for t in ts]
                    ads = [i * L + lane for i in ids]
                    for a in ads:
                        plsc.addupdate_scatter(hist, [a], ones_v)
                    qs = [ids[4 * k] | (ids[4 * k + 1] << 8)
                          | (ids[4 * k + 2] << 16) | (ids[4 * k + 3] << 24)
                          for k in range(G // 4)]
                    base = ((c * RPC + r) * w + g * G * L) // 4
                    for k, q in enumerate(qs):
                        idxc[pl.ds(base + k * L, L)] = q

            return carry

        stream_in(img, hist_chunk, 0)

        def grp(j2, tot):
            base = j2 * (L * L)
            acc = zero_v
            for k in range(L):
                acc = acc + plsc.load_gather(hist, [base + lane * L + k])
            c = plsc.cumsum(acc) + tot
            cdf[pl.ds(j2 * L, L)] = c
            return c[L - 1]

        tot = lax.fori_loop(0, BINS // L, grp, jnp.float32(0.0))
        c0 = cdf[pl.ds(0, L)][0]
        inv = (jnp.ones((L,), jnp.float32) / (tot - c0 + 1e-8))[0]

        @plsc.parallel_loop(0, BINS, step=L)
        def nrm(j2):
            v = cdf[pl.ds(j2, L)]
            cdf[pl.ds(j2, L)] = (v - c0) * inv

        n_oc = rows // OPC

        def eq_outer(c2, _):
            for b in range(2):
                c = c2 * 2 + b
                ob = obufs[b]

                @pl.when(c2 > 0)
                def _():
                    pltpu.make_async_copy(
                        ob, out_hbm.at[img, pl.ds(c * OPC, OPC)],
                        osems[b]).wait()

                @plsc.parallel_loop(0, OPC)
                def vec_e(r):
                    base = (c * OPC + r) * w // 4
                    qvs = [idxc[pl.ds(base + q * L, L)]
                           for q in range(vpr // 4)]
                    m255 = jnp.int32(255)
                    for q, qv in enumerate(qvs):
                        ivs = [qv & m255,
                               lax.shift_right_logical(qv, 8) & m255,
                               lax.shift_right_logical(qv, 16) & m255,
                               lax.shift_right_logical(qv, 24)]
                        for k, idx in enumerate(ivs):
                            ob[r, pl.ds((q * 4 + k) * L, L)] = (
                                plsc.load_gather(cdf, [idx]))

                pltpu.async_copy(ob, out_hbm.at[img, pl.ds(c * OPC, OPC)],
                                 osems[b])
            return 0

        lax.fori_loop(0, n_oc // 2, eq_outer, 0)
        for b in range(2):
            pltpu.make_async_copy(
                obufs[b],
                out_hbm.at[img, pl.ds((n_oc - 2 + b) * OPC, OPC)],
                osems[b]).wait()


def kernel(x):
    b, h, w = x.shape
    mesh = plsc.VectorSubcoreMesh(core_axis_name="c", subcore_axis_name="s")
    run = pl.kernel(
        _body,
        out_type=jax.ShapeDtypeStruct((b, h, w), jnp.float32),
        mesh=mesh,
        compiler_params=pltpu.CompilerParams(
            needs_layout_passes=False, use_tc_tiling_on_sc=True),
        scratch_types=[
            pltpu.VMEM((RPC, w), jnp.float32),
            pltpu.VMEM((RPC, w), jnp.float32),
            pltpu.VMEM((OPC, w), jnp.float32),
            pltpu.VMEM((OPC, w), jnp.float32),
            pltpu.VMEM((h * w // 4,), jnp.int32),
            pltpu.VMEM((BINS * L,), jnp.float32),
            pltpu.VMEM((BINS,), jnp.float32),
            pltpu.SemaphoreType.DMA,
            pltpu.SemaphoreType.DMA,
            pltpu.SemaphoreType.DMA,
            pltpu.SemaphoreType.DMA,
        ],
    )
    return run(x)


# speedup vs baseline: 1.2600x; 1.0459x over previous
import jax
import jax.numpy as jnp
from jax import lax
from jax.experimental import pallas as pl
from jax.experimental.pallas import tpu as pltpu
from jax.experimental.pallas import tpu_sc as plsc

BINS = 256
NC = 2
NS = 16
L = 16
NW = NC * NS
RPC = 32
OPC = 16
G = 16


def _body(x_hbm, out_hbm, xbuf0, xbuf1, obuf0, obuf1, idxc, hist, cdf,
          sem0, sem1, osem0, osem1):
    wid = lax.axis_index("s") * NC + lax.axis_index("c")
    lane = lax.iota(jnp.int32, L)
    lane_off = lane * BINS
    rows = x_hbm.shape[1]
    w = x_hbm.shape[2]
    vpr = w // L
    n_chunks = rows // RPC
    bufs = (xbuf0, xbuf1)
    sems = (sem0, sem1)
    obufs = (obuf0, obuf1)
    osems = (osem0, osem1)

    def stream_in(img, compute, carry_init):
        pltpu.async_copy(x_hbm.at[img, pl.ds(0, RPC)], bufs[0], sems[0])

        def outer(c2, carry):
            for b in range(2):
                c = c2 * 2 + b
                nb = (b + 1) % 2

                @pl.when(c + 1 < n_chunks)
                def _():
                    pltpu.async_copy(
                        x_hbm.at[img, pl.ds((c + 1) * RPC, RPC)],
                        bufs[nb], sems[nb])

                pltpu.make_async_copy(
                    x_hbm.at[img, pl.ds(c * RPC, RPC)], bufs[b], sems[b]).wait()
                carry = compute(c, bufs[b], carry)
            return carry

        return lax.fori_loop(0, n_chunks // 2, outer, carry_init)

    for rr in range(x_hbm.shape[0] // NW):
        img = wid * (x_hbm.shape[0] // NW) + rr

        def mm_chunk(c, buf, carry):
            @plsc.parallel_loop(0, RPC, carry=carry)
            def mm(r, carry2):
                vmn2, vmx2 = carry2
                vs = [buf[r, pl.ds(k * L, L)] for k in range(vpr)]
                lo, hi = vs, vs
                while len(lo) > 1:
                    lo = [jnp.minimum(a, b) for a, b in zip(lo[::2], lo[1::2])]
                    hi = [jnp.maximum(a, b) for a, b in zip(hi[::2], hi[1::2])]
                return (jnp.minimum(vmn2, lo[0]), jnp.maximum(vmx2, hi[0]))

            return mm

        vmn0 = jnp.full((L,), jnp.inf, jnp.float32)
        vmx0 = jnp.full((L,), -jnp.inf, jnp.float32)
        vmn, vmx = stream_in(img, mm_chunk, (vmn0, vmx0))
        mn = -plsc.cummax(-vmn)[L - 1]
        mx = plsc.cummax(vmx)[L - 1]
        scale_v = jnp.full((L,), float(BINS - 1), jnp.float32) / (mx - mn + 1e-8)
        scale = scale_v[0]

        zero_v = jnp.zeros((L,), jnp.float32)

        @plsc.parallel_loop(0, BINS * L, step=L)
        def zrow(j):
            hist[pl.ds(j, L)] = zero_v

        ones_v = jnp.ones((L,), jnp.float32)

        def hist_chunk(c, buf, carry):
            @plsc.parallel_loop(0, RPC)
            def vec_h(r):
                for g in range(vpr // G):
                    vs = [buf[r, pl.ds((g * G + k) * L, L)] for k in range(G)]
                    ts = [v - mn for v in vs]
                    ts = [t * scale for t in ts]
                    ts = [jnp.minimum(t, float(BINS - 1)) for t in ts]
                    ids = [t.astype(jnp.int32) for t in ts]
                    ads = [i * L + lane for i in ids]
                    for a in ads:
                        plsc.addupdate_scatter(hist, [a], ones_v)
                    qs = [ids[4 * k] | (ids[4 * k + 1] << 8)
                          | (ids[4 * k + 2] << 16) | (ids[4 * k + 3] << 24)
                          for k in range(G // 4)]
                    base = ((c * RPC + r) * w + g * G * L) // 4
                    for k, q in enumerate(qs):
                        idxc[pl.ds(base + k * L, L)] = q

            return carry

        stream_in(img, hist_chunk, 0)

        def grp(j2, tot):
            base = j2 * (L * L)
            acc = zero_v
            for k in range(L):
                acc = acc + plsc.load_gather(hist, [base + lane * L + k])
            c = plsc.cumsum(acc) + tot
            cdf[pl.ds(j2 * L, L)] = c
            return c[L - 1]

        tot = lax.fori_loop(0, BINS // L, grp, jnp.float32(0.0))
        c0 = cdf[pl.ds(0, L)][0]
        inv = (jnp.ones((L,), jnp.float32) / (tot - c0 + 1e-8))[0]

        @plsc.parallel_loop(0, BINS, step=L)
        def nrm(j2):
            v = cdf[pl.ds(j2, L)]
            cdf[pl.ds(j2, L)] = (v - c0) * inv

        n_oc = rows // OPC

        def eq_outer(c2, _):
            for b in range(2):
                c = c2 * 2 + b
                ob = obufs[b]

                @pl.when(c2 > 0)
                def _():
                    pltpu.make_async_copy(
                        ob, out_hbm.at[img, pl.ds(c * OPC, OPC)],
                        osems[b]).wait()

                @plsc.parallel_loop(0, OPC)
                def vec_e(r):
                    base = (c * OPC + r) * w // 4
                    qvs = [idxc[pl.ds(base + q * L, L)]
                           for q in range(vpr // 4)]
                    m255 = jnp.int32(255)
                    for q, qv in enumerate(qvs):
                        ivs = [qv & m255,
                               lax.shift_right_logical(qv, 8) & m255,
                               lax.shift_right_logical(qv, 16) & m255,
                               lax.shift_right_logical(qv, 24)]
                        for k, idx in enumerate(ivs):
                            ob[r, pl.ds((q * 4 + k) * L, L)] = (
                                plsc.load_gather(cdf, [idx]))

                pltpu.async_copy(ob, out_hbm.at[img, pl.ds(c * OPC, OPC)],
                                 osems[b])
            return 0

        lax.fori_loop(0, n_oc // 2, eq_outer, 0)
        for b in range(2):
            pltpu.make_async_copy(
                obufs[b],
                out_hbm.at[img, pl.ds((n_oc - 2 + b) * OPC, OPC)],
                osems[b]).wait()


def kernel(x):
    b, h, w = x.shape
    mesh = plsc.VectorSubcoreMesh(core_axis_name="c", subcore_axis_name="s")
    run = pl.kernel(
        _body,
        out_type=jax.ShapeDtypeStruct((b, h, w), jnp.float32),
        mesh=mesh,
        compiler_params=pltpu.CompilerParams(
            needs_layout_passes=False, use_tc_tiling_on_sc=True),
        scratch_types=[
            pltpu.VMEM((RPC, w), jnp.float32),
            pltpu.VMEM((RPC, w), jnp.float32),
            pltpu.VMEM((OPC, w), jnp.float32),
            pltpu.VMEM((OPC, w), jnp.float32),
            pltpu.VMEM((h * w // 4,), jnp.int32),
            pltpu.VMEM((BINS * L,), jnp.float32),
            pltpu.VMEM((BINS,), jnp.float32),
            pltpu.SemaphoreType.DMA,
            pltpu.SemaphoreType.DMA,
            pltpu.SemaphoreType.DMA,
            pltpu.SemaphoreType.DMA,
        ],
    )
    return run(x)


# speedup vs baseline: 1.2738x; 1.0109x over previous
import jax
import jax.numpy as jnp
from jax import lax
from jax.experimental import pallas as pl
from jax.experimental.pallas import tpu as pltpu
from jax.experimental.pallas import tpu_sc as plsc

BINS = 256
NC = 2
NS = 16
L = 16
NW = NC * NS
RPC = 32
OPC = 16
G = 32


def _body(x_hbm, out_hbm, xbuf0, xbuf1, obuf0, obuf1, idxc, hist, cdf,
          sem0, sem1, osem0, osem1):
    wid = lax.axis_index("s") * NC + lax.axis_index("c")
    lane = lax.iota(jnp.int32, L)
    lane_off = lane * BINS
    rows = x_hbm.shape[1]
    w = x_hbm.shape[2]
    vpr = w // L
    n_chunks = rows // RPC
    bufs = (xbuf0, xbuf1)
    sems = (sem0, sem1)
    obufs = (obuf0, obuf1)
    osems = (osem0, osem1)

    def stream_in(img, compute, carry_init):
        pltpu.async_copy(x_hbm.at[img, pl.ds(0, RPC)], bufs[0], sems[0])

        def outer(c2, carry):
            for b in range(2):
                c = c2 * 2 + b
                nb = (b + 1) % 2

                @pl.when(c + 1 < n_chunks)
                def _():
                    pltpu.async_copy(
                        x_hbm.at[img, pl.ds((c + 1) * RPC, RPC)],
                        bufs[nb], sems[nb])

                pltpu.make_async_copy(
                    x_hbm.at[img, pl.ds(c * RPC, RPC)], bufs[b], sems[b]).wait()
                carry = compute(c, bufs[b], carry)
            return carry

        return lax.fori_loop(0, n_chunks // 2, outer, carry_init)

    for rr in range(x_hbm.shape[0] // NW):
        img = wid * (x_hbm.shape[0] // NW) + rr

        def mm_chunk(c, buf, carry):
            @plsc.parallel_loop(0, RPC, carry=carry)
            def mm(r, carry2):
                vmn2, vmx2 = carry2
                vs = [buf[r, pl.ds(k * L, L)] for k in range(vpr)]
                lo, hi = vs, vs
                while len(lo) > 1:
                    lo = [jnp.minimum(a, b) for a, b in zip(lo[::2], lo[1::2])]
                    hi = [jnp.maximum(a, b) for a, b in zip(hi[::2], hi[1::2])]
                return (jnp.minimum(vmn2, lo[0]), jnp.maximum(vmx2, hi[0]))

            return mm

        vmn0 = jnp.full((L,), jnp.inf, jnp.float32)
        vmx0 = jnp.full((L,), -jnp.inf, jnp.float32)
        vmn, vmx = stream_in(img, mm_chunk, (vmn0, vmx0))
        mn = -plsc.cummax(-vmn)[L - 1]
        mx = plsc.cummax(vmx)[L - 1]
        scale_v = jnp.full((L,), float(BINS - 1), jnp.float32) / (mx - mn + 1e-8)
        scale = scale_v[0]

        zero_v = jnp.zeros((L,), jnp.float32)

        @plsc.parallel_loop(0, BINS * L, step=L)
        def zrow(j):
            hist[pl.ds(j, L)] = zero_v

        ones_v = jnp.ones((L,), jnp.float32)

        def hist_chunk(c, buf, carry):
            @plsc.parallel_loop(0, RPC)
            def vec_h(r):
                for g in range(vpr // G):
                    vs = [buf[r, pl.ds((g * G + k) * L, L)] for k in range(G)]
                    ts = [v - mn for v in vs]
                    ts = [t * scale for t in ts]
                    ts = [jnp.minimum(t, float(BINS - 1)) for t in ts]
                    ids = [t.astype(jnp.int32) for t in ts]
                    ads = [i * L + lane for i in ids]
                    for a in ads:
                        plsc.addupdate_scatter(hist, [a], ones_v)
                    qs = [ids[4 * k] | (ids[4 * k + 1] << 8)
                          | (ids[4 * k + 2] << 16) | (ids[4 * k + 3] << 24)
                          for k in range(G // 4)]
                    base = ((c * RPC + r) * w + g * G * L) // 4
                    for k, q in enumerate(qs):
                        idxc[pl.ds(base + k * L, L)] = q

            return carry

        stream_in(img, hist_chunk, 0)

        def grp(j2, tot):
            base = j2 * (L * L)
            acc = zero_v
            for k in range(L):
                acc = acc + plsc.load_gather(hist, [base + lane * L + k])
            c = plsc.cumsum(acc) + tot
            cdf[pl.ds(j2 * L, L)] = c
            return c[L - 1]

        tot = lax.fori_loop(0, BINS // L, grp, jnp.float32(0.0))
        c0 = cdf[pl.ds(0, L)][0]
        inv = (jnp.ones((L,), jnp.float32) / (tot - c0 + 1e-8))[0]

        @plsc.parallel_loop(0, BINS, step=L)
        def nrm(j2):
            v = cdf[pl.ds(j2, L)]
            cdf[pl.ds(j2, L)] = (v - c0) * inv

        n_oc = rows // OPC

        def eq_outer(c2, _):
            for b in range(2):
                c = c2 * 2 + b
                ob = obufs[b]

                @pl.when(c2 > 0)
                def _():
                    pltpu.make_async_copy(
                        ob, out_hbm.at[img, pl.ds(c * OPC, OPC)],
                        osems[b]).wait()

                @plsc.parallel_loop(0, OPC)
                def vec_e(r):
                    base = (c * OPC + r) * w // 4
                    qvs = [idxc[pl.ds(base + q * L, L)]
                           for q in range(vpr // 4)]
                    m255 = jnp.int32(255)
                    for q, qv in enumerate(qvs):
                        ivs = [qv & m255,
                               lax.shift_right_logical(qv, 8) & m255,
                               lax.shift_right_logical(qv, 16) & m255,
                               lax.shift_right_logical(qv, 24)]
                        for k, idx in enumerate(ivs):
                            ob[r, pl.ds((q * 4 + k) * L, L)] = (
                                plsc.load_gather(cdf, [idx]))

                pltpu.async_copy(ob, out_hbm.at[img, pl.ds(c * OPC, OPC)],
                                 osems[b])
            return 0

        lax.fori_loop(0, n_oc // 2, eq_outer, 0)
        for b in range(2):
            pltpu.make_async_copy(
                obufs[b],
                out_hbm.at[img, pl.ds((n_oc - 2 + b) * OPC, OPC)],
                osems[b]).wait()


def kernel(x):
    b, h, w = x.shape
    mesh = plsc.VectorSubcoreMesh(core_axis_name="c", subcore_axis_name="s")
    run = pl.kernel(
        _body,
        out_type=jax.ShapeDtypeStruct((b, h, w), jnp.float32),
        mesh=mesh,
        compiler_params=pltpu.CompilerParams(
            needs_layout_passes=False, use_tc_tiling_on_sc=True),
        scratch_types=[
            pltpu.VMEM((RPC, w), jnp.float32),
            pltpu.VMEM((RPC, w), jnp.float32),
            pltpu.VMEM((OPC, w), jnp.float32),
            pltpu.VMEM((OPC, w), jnp.float32),
            pltpu.VMEM((h * w // 4,), jnp.int32),
            pltpu.VMEM((BINS * L,), jnp.float32),
            pltpu.VMEM((BINS,), jnp.float32),
            pltpu.SemaphoreType.DMA,
            pltpu.SemaphoreType.DMA,
            pltpu.SemaphoreType.DMA,
            pltpu.SemaphoreType.DMA,
        ],
    )
    return run(x)


# speedup vs baseline: 1.2754x; 1.0013x over previous
import jax
import jax.numpy as jnp
from jax import lax
from jax.experimental import pallas as pl
from jax.experimental.pallas import tpu as pltpu
from jax.experimental.pallas import tpu_sc as plsc

BINS = 256
NC = 2
NS = 16
L = 16
NW = NC * NS
RPC = 32
OPC = 16
G = 32


def _body(x_hbm, out_hbm, xbuf0, xbuf1, obuf0, obuf1, idxc, hist, cdf,
          sem0, sem1, osem0, osem1):
    wid = lax.axis_index("s") * NC + lax.axis_index("c")
    lane = lax.iota(jnp.int32, L)
    rows = x_hbm.shape[1]
    w = x_hbm.shape[2]
    vpr = w // L
    n_chunks = rows // RPC
    bufs = (xbuf0, xbuf1)
    sems = (sem0, sem1)
    obufs = (obuf0, obuf1)
    osems = (osem0, osem1)

    def stream_in(img, compute, carry_init):
        pltpu.async_copy(x_hbm.at[img, pl.ds(0, RPC)], bufs[0], sems[0])

        def outer(c2, carry):
            for b in range(2):
                c = c2 * 2 + b
                nb = (b + 1) % 2

                @pl.when(c + 1 < n_chunks)
                def _():
                    pltpu.async_copy(
                        x_hbm.at[img, pl.ds((c + 1) * RPC, RPC)],
                        bufs[nb], sems[nb])

                pltpu.make_async_copy(
                    x_hbm.at[img, pl.ds(c * RPC, RPC)], bufs[b], sems[b]).wait()
                carry = compute(c, bufs[b], carry)
            return carry

        return lax.fori_loop(0, n_chunks // 2, outer, carry_init)

    for rr in range(x_hbm.shape[0] // NW):
        img = wid * (x_hbm.shape[0] // NW) + rr

        def mm_chunk(c, buf, carry):
            @plsc.parallel_loop(0, RPC, carry=carry)
            def mm(r, carry2):
                vmn2, vmx2 = carry2
                vs = [buf[r, pl.ds(k * L, L)] for k in range(vpr)]
                lo, hi = vs, vs
                while len(lo) > 1:
                    lo = [jnp.minimum(a, b) for a, b in zip(lo[::2], lo[1::2])]
                    hi = [jnp.maximum(a, b) for a, b in zip(hi[::2], hi[1::2])]
                return (jnp.minimum(vmn2, lo[0]), jnp.maximum(vmx2, hi[0]))

            return mm

        vmn0 = jnp.full((L,), jnp.inf, jnp.float32)
        vmx0 = jnp.full((L,), -jnp.inf, jnp.float32)
        vmn, vmx = stream_in(img, mm_chunk, (vmn0, vmx0))
        mn = -plsc.cummax(-vmn)[L - 1]
        mx = plsc.cummax(vmx)[L - 1]
        scale_v = jnp.full((L,), float(BINS - 1), jnp.float32) / (mx - mn + 1e-8)
        scale = scale_v[0]

        zero_v = jnp.zeros((L,), jnp.float32)

        @plsc.parallel_loop(0, BINS * L, step=L)
        def zrow(j):
            hist[pl.ds(j, L)] = zero_v

        ones_v = jnp.ones((L,), jnp.float32)

        def hist_chunk(c, buf, carry):
            @plsc.parallel_loop(0, RPC)
            def vec_h(r):
                for g in range(vpr // G):
                    vs = [buf[r, pl.ds((g * G + k) * L, L)] for k in range(G)]
                    ts = [v - mn for v in vs]
                    ts = [t * scale for t in ts]
                    ts = [jnp.minimum(t, float(BINS - 1)) for t in ts]
                    ids = [t.astype(jnp.int32) for t in ts]
                    ads = [i * L + lane for i in ids]
                    for a in ads:
                        plsc.addupdate_scatter(hist, [a], ones_v)
                    qs = [ids[4 * k] | (ids[4 * k + 1] << 8)
                          | (ids[4 * k + 2] << 16) | (ids[4 * k + 3] << 24)
                          for k in range(G // 4)]
                    base = ((c * RPC + r) * w + g * G * L) // 4
                    for k, q in enumerate(qs):
                        idxc[pl.ds(base + k * L, L)] = q

            return carry

        stream_in(img, hist_chunk, 0)

        def grp(j2, tot):
            base = j2 * (L * L)
            acc = zero_v
            for k in range(L):
                acc = acc + plsc.load_gather(hist, [base + lane * L + k])
            c = plsc.cumsum(acc) + tot
            cdf[pl.ds(j2 * L, L)] = c
            return c[L - 1]

        tot = lax.fori_loop(0, BINS // L, grp, jnp.float32(0.0))
        c0 = cdf[pl.ds(0, L)][0]
        inv = (jnp.ones((L,), jnp.float32) / (tot - c0 + 1e-8))[0]

        @plsc.parallel_loop(0, BINS, step=L)
        def nrm(j2):
            v = cdf[pl.ds(j2, L)]
            cdf[pl.ds(j2, L)] = (v - c0) * inv

        n_oc = rows // OPC

        def eq_outer(c2, _):
            for b in range(2):
                c = c2 * 2 + b
                ob = obufs[b]

                @pl.when(c2 > 0)
                def _():
                    pltpu.make_async_copy(
                        ob, out_hbm.at[img, pl.ds(c * OPC, OPC)],
                        osems[b]).wait()

                @plsc.parallel_loop(0, OPC)
                def vec_e(r):
                    base = (c * OPC + r) * w // 4
                    qvs = [idxc[pl.ds(base + q * L, L)]
                           for q in range(vpr // 4)]
                    m255 = jnp.int32(255)
                    for q, qv in enumerate(qvs):
                        ivs = [qv & m255,
                               lax.shift_right_logical(qv, 8) & m255,
                               lax.shift_right_logical(qv, 16) & m255,
                               lax.shift_right_logical(qv, 24)]
                        for k, idx in enumerate(ivs):
                            ob[r, pl.ds((q * 4 + k) * L, L)] = (
                                plsc.load_gather(cdf, [idx]))

                pltpu.async_copy(ob, out_hbm.at[img, pl.ds(c * OPC, OPC)],
                                 osems[b])
            return 0

        lax.fori_loop(0, n_oc // 2, eq_outer, 0)
        for b in range(2):
            pltpu.make_async_copy(
                obufs[b],
                out_hbm.at[img, pl.ds((n_oc - 2 + b) * OPC, OPC)],
                osems[b]).wait()


def kernel(x):
    b, h, w = x.shape
    mesh = plsc.VectorSubcoreMesh(core_axis_name="c", subcore_axis_name="s")
    run = pl.kernel(
        _body,
        out_type=jax.ShapeDtypeStruct((b, h, w), jnp.float32),
        mesh=mesh,
        compiler_params=pltpu.CompilerParams(
            needs_layout_passes=False, use_tc_tiling_on_sc=True),
        scratch_types=[
            pltpu.VMEM((RPC, w), jnp.float32),
            pltpu.VMEM((RPC, w), jnp.float32),
            pltpu.VMEM((OPC, w), jnp.float32),
            pltpu.VMEM((OPC, w), jnp.float32),
            pltpu.VMEM((h * w // 4,), jnp.int32),
            pltpu.VMEM((BINS * L,), jnp.float32),
            pltpu.VMEM((BINS,), jnp.float32),
            pltpu.SemaphoreType.DMA,
            pltpu.SemaphoreType.DMA,
            pltpu.SemaphoreType.DMA,
            pltpu.SemaphoreType.DMA,
        ],
    )
    return run(x)
